# baseline (device time: 82287 ns/iter reference)
import jax
import jax.numpy as jnp
from jax import lax
from jax.experimental import pallas as pl
from jax.experimental.pallas import tpu as pltpu

N_DEV = 32
BLK = 128
K_HALVES = 2
HALF = N_DEV // K_HALVES
K_STEP = HALF * BLK
N_QUART = 4
N_TILE = 8192 // N_QUART


def kernel(x, w_mat):
    K, kcols = x.shape
    _, N = w_mat.shape
    assert kcols == BLK and K == N_DEV * BLK

    def body(order_ref, x_ref, w_ref, out_ref, recv_buf, a_stage, y_acc,
             amax_tx, amax_buf, send_sems, recv_sems, a_send_sems,
             a_recv_sems):
        k = pl.program_id(0)
        n = pl.program_id(1)
        my = lax.axis_index("i")
        my_sb = my // HALF
        my_off = my % HALF

        def stagger_target(i):
            g = i // HALF
            c = i % HALF
            return HALF * ((my_sb + g) % K_HALVES) + ((my_off + 1 + c) % HALF)

        @pl.when((k == 0) & (n == 0))
        def _():
            barrier = pltpu.get_barrier_semaphore()

            def sig_body(j, _):
                @pl.when(j != my)
                def _():
                    pl.semaphore_signal(
                        barrier, inc=1,
                        device_id=(j,), device_id_type=pl.DeviceIdType.MESH,
                    )
                return 0

            lax.fori_loop(0, N_DEV, sig_body, 0)
            pl.semaphore_wait(barrier, N_DEV - 1)
            amax_buf[...] = jnp.zeros_like(amax_buf)

            def send_body(i, _):
                r = stagger_target(i)

                @pl.when(r != my)
                def _():
                    pltpu.make_async_remote_copy(
                        src_ref=x_ref.at[pl.ds(r * BLK, BLK), :],
                        dst_ref=recv_buf.at[my],
                        send_sem=send_sems.at[r],
                        recv_sem=recv_sems.at[my],
                        device_id=(r,),
                        device_id_type=pl.DeviceIdType.MESH,
                    ).start()
                return 0

            lax.fori_loop(0, N_DEV, send_body, 0)

        s = order_ref[k]

        @pl.when(n == 0)
        def _():
            def recv_body(c, _):
                b = s * HALF + c

                @pl.when(b != my)
                def _():
                    pltpu.make_async_remote_copy(
                        src_ref=x_ref.at[pl.ds(0, BLK), :],
                        dst_ref=recv_buf.at[b],
                        send_sem=send_sems.at[0],
                        recv_sem=recv_sems.at[b],
                        device_id=(0,),
                        device_id_type=pl.DeviceIdType.MESH,
                    ).wait_recv()
                return 0

            lax.fori_loop(0, HALF, recv_body, 0)

            a_own = x_ref[pl.ds(my * BLK, BLK), :]
            blocks = []
            for c in range(HALF):
                b = s * HALF + c
                blocks.append(jnp.where(b == my, a_own, recv_buf[b]))
            a_stage[...] = jnp.concatenate(blocks, axis=1)

        contrib = jnp.dot(
            a_stage[...], w_ref[...], preferred_element_type=jnp.float32
        )

        @pl.when(k == 0)
        def _():
            y_acc[n] = contrib

        @pl.when(k > 0)
        def _():
            y_acc[n] = y_acc[n] + contrib

        @pl.when((k == K_HALVES - 1) & (n == N_QUART - 1))
        def _():
            local_amax = jnp.max(jnp.abs(y_acc[...]))
            amax_tx[...] = jnp.full((1, BLK), local_amax, jnp.float32)

            def amax_send_body(i, _):
                r = stagger_target(i)

                @pl.when(r != my)
                def _():
                    pltpu.make_async_remote_copy(
                        src_ref=amax_tx,
                        dst_ref=amax_buf.at[pl.ds(my, 1), :],
                        send_sem=a_send_sems.at[r],
                        recv_sem=a_recv_sems.at[my],
                        device_id=(r,),
                        device_id_type=pl.DeviceIdType.MESH,
                    ).start()
                return 0

            lax.fori_loop(0, N_DEV, amax_send_body, 0)

            def amax_wait_body(j, _):
                @pl.when(j != my)
                def _():
                    pltpu.make_async_remote_copy(
                        src_ref=amax_tx,
                        dst_ref=amax_buf.at[pl.ds(j, 1), :],
                        send_sem=a_send_sems.at[j],
                        recv_sem=a_recv_sems.at[j],
                        device_id=(0,),
                        device_id_type=pl.DeviceIdType.MESH,
                    ).wait_recv()
                return 0

            lax.fori_loop(0, N_DEV, amax_wait_body, 0)

            g_amax = jnp.maximum(jnp.max(amax_buf[...]), local_amax)
            scale = g_amax / 127.0
            for i in range(N_QUART):
                q = jnp.clip(jnp.round(y_acc[i] / scale), -127.0, 127.0)
                out_ref[:, i * N_TILE:(i + 1) * N_TILE] = q * scale

            def drain_body(j, _):
                @pl.when(j != my)
                def _():
                    pltpu.make_async_remote_copy(
                        src_ref=x_ref.at[pl.ds(0, BLK), :],
                        dst_ref=recv_buf.at[my],
                        send_sem=send_sems.at[j],
                        recv_sem=recv_sems.at[my],
                        device_id=(j,),
                        device_id_type=pl.DeviceIdType.MESH,
                    ).wait_send()
                    pltpu.make_async_remote_copy(
                        src_ref=amax_tx,
                        dst_ref=amax_buf.at[pl.ds(my, 1), :],
                        send_sem=a_send_sems.at[j],
                        recv_sem=a_recv_sems.at[my],
                        device_id=(j,),
                        device_id_type=pl.DeviceIdType.MESH,
                    ).wait_send()
                return 0

            lax.fori_loop(0, N_DEV, drain_body, 0)

    my_sb = lax.axis_index("i") // HALF
    order = jnp.mod(my_sb + jnp.arange(K_HALVES, dtype=jnp.int32), K_HALVES)

    grid_spec = pltpu.PrefetchScalarGridSpec(
        num_scalar_prefetch=1,
        grid=(K_HALVES, N_QUART),
        in_specs=[
            pl.BlockSpec((K, BLK), lambda k, n, order: (0, 0)),
            pl.BlockSpec(
                (K_STEP, N_TILE), lambda k, n, order: (order[k], n)
            ),
        ],
        out_specs=pl.BlockSpec((BLK, N), lambda k, n, order: (0, 0)),
        scratch_shapes=[
            pltpu.VMEM((N_DEV, BLK, BLK), jnp.float32),
            pltpu.VMEM((BLK, K_STEP), jnp.float32),
            pltpu.VMEM((N_QUART, BLK, N_TILE), jnp.float32),
            pltpu.VMEM((1, BLK), jnp.float32),
            pltpu.VMEM((N_DEV, BLK), jnp.float32),
            pltpu.SemaphoreType.DMA((N_DEV,)),
            pltpu.SemaphoreType.DMA((N_DEV,)),
            pltpu.SemaphoreType.DMA((N_DEV,)),
            pltpu.SemaphoreType.DMA((N_DEV,)),
        ],
    )

    return pl.pallas_call(
        body,
        grid_spec=grid_spec,
        out_shape=jax.ShapeDtypeStruct((BLK, N), jnp.float32),
        compiler_params=pltpu.CompilerParams(
            collective_id=0,
            vmem_limit_bytes=56 * 1024 * 1024,
        ),
    )(order, x, w_mat)


# device time: 76499 ns/iter; 1.0757x vs baseline; 1.0757x over previous
import jax
import jax.numpy as jnp
from jax import lax
from jax.experimental import pallas as pl
from jax.experimental.pallas import tpu as pltpu

N_DEV = 32
BLK = 128
QUARTERS = 4
QB = N_DEV // QUARTERS
K_STEP = QB * BLK
N_HALVES = 2
N_TILE = 8192 // N_HALVES


def kernel(x, w_mat):
    K, kcols = x.shape
    _, N = w_mat.shape
    assert kcols == BLK and K == N_DEV * BLK

    def body(order_ref, x_ref, w_ref, out_ref, recv_buf, a_stage, y_acc,
             amax_tx, amax_buf, send_sems, recv_sems, a_send_sems,
             a_recv_sems):
        k = pl.program_id(0)
        n = pl.program_id(1)
        my = lax.axis_index("i")
        my_qb = my // QB
        my_off = my % QB

        def stagger_target(i):
            g = i // QB
            c = i % QB
            return QB * jnp.mod(my_qb - g, QUARTERS) + ((my_off + 1 + c) % QB)

        @pl.when((k == 0) & (n == 0))
        def _():
            barrier = pltpu.get_barrier_semaphore()

            def sig_body(j, _):
                @pl.when(j != my)
                def _():
                    pl.semaphore_signal(
                        barrier, inc=1,
                        device_id=(j,), device_id_type=pl.DeviceIdType.MESH,
                    )
                return 0

            lax.fori_loop(0, N_DEV, sig_body, 0)
            pl.semaphore_wait(barrier, N_DEV - 1)
            amax_buf[...] = jnp.zeros_like(amax_buf)

            def send_body(i, _):
                r = stagger_target(i)

                @pl.when(r != my)
                def _():
                    pltpu.make_async_remote_copy(
                        src_ref=x_ref.at[pl.ds(r * BLK, BLK), :],
                        dst_ref=recv_buf.at[my],
                        send_sem=send_sems.at[r],
                        recv_sem=recv_sems.at[my],
                        device_id=(r,),
                        device_id_type=pl.DeviceIdType.MESH,
                    ).start()
                return 0

            lax.fori_loop(0, N_DEV, send_body, 0)

        s = order_ref[k]

        @pl.when(n == 0)
        def _():
            def recv_body(c, _):
                b = s * QB + c

                @pl.when(b != my)
                def _():
                    pltpu.make_async_remote_copy(
                        src_ref=x_ref.at[pl.ds(0, BLK), :],
                        dst_ref=recv_buf.at[b],
                        send_sem=send_sems.at[0],
                        recv_sem=recv_sems.at[b],
                        device_id=(0,),
                        device_id_type=pl.DeviceIdType.MESH,
                    ).wait_recv()
                return 0

            lax.fori_loop(0, QB, recv_body, 0)

            a_own = x_ref[pl.ds(my * BLK, BLK), :]
            blocks = []
            for c in range(QB):
                b = s * QB + c
                blocks.append(jnp.where(b == my, a_own, recv_buf[b]))
            a_stage[...] = jnp.concatenate(blocks, axis=1)

        contrib = jnp.dot(
            a_stage[...], w_ref[...], preferred_element_type=jnp.float32
        )

        @pl.when(k == 0)
        def _():
            y_acc[n] = contrib

        @pl.when(k > 0)
        def _():
            y_acc[n] = y_acc[n] + contrib

        @pl.when((k == QUARTERS - 1) & (n == N_HALVES - 1))
        def _():
            local_amax = jnp.max(jnp.abs(y_acc[...]))
            amax_tx[...] = jnp.full((1, BLK), local_amax, jnp.float32)

            def amax_send_body(i, _):
                r = stagger_target(i)

                @pl.when(r != my)
                def _():
                    pltpu.make_async_remote_copy(
                        src_ref=amax_tx,
                        dst_ref=amax_buf.at[pl.ds(my, 1), :],
                        send_sem=a_send_sems.at[r],
                        recv_sem=a_recv_sems.at[my],
                        device_id=(r,),
                        device_id_type=pl.DeviceIdType.MESH,
                    ).start()
                return 0

            lax.fori_loop(0, N_DEV, amax_send_body, 0)

            def amax_wait_body(j, _):
                @pl.when(j != my)
                def _():
                    pltpu.make_async_remote_copy(
                        src_ref=amax_tx,
                        dst_ref=amax_buf.at[pl.ds(j, 1), :],
                        send_sem=a_send_sems.at[j],
                        recv_sem=a_recv_sems.at[j],
                        device_id=(0,),
                        device_id_type=pl.DeviceIdType.MESH,
                    ).wait_recv()
                return 0

            lax.fori_loop(0, N_DEV, amax_wait_body, 0)

            g_amax = jnp.maximum(jnp.max(amax_buf[...]), local_amax)
            scale = g_amax / 127.0
            for i in range(N_HALVES):
                q = jnp.clip(jnp.round(y_acc[i] / scale), -127.0, 127.0)
                out_ref[:, i * N_TILE:(i + 1) * N_TILE] = q * scale

            def drain_body(j, _):
                @pl.when(j != my)
                def _():
                    pltpu.make_async_remote_copy(
                        src_ref=x_ref.at[pl.ds(0, BLK), :],
                        dst_ref=recv_buf.at[my],
                        send_sem=send_sems.at[j],
                        recv_sem=recv_sems.at[my],
                        device_id=(j,),
                        device_id_type=pl.DeviceIdType.MESH,
                    ).wait_send()
                    pltpu.make_async_remote_copy(
                        src_ref=amax_tx,
                        dst_ref=amax_buf.at[pl.ds(my, 1), :],
                        send_sem=a_send_sems.at[j],
                        recv_sem=a_recv_sems.at[my],
                        device_id=(j,),
                        device_id_type=pl.DeviceIdType.MESH,
                    ).wait_send()
                return 0

            lax.fori_loop(0, N_DEV, drain_body, 0)

    my_qb = lax.axis_index("i") // QB
    order = jnp.mod(my_qb + jnp.arange(QUARTERS, dtype=jnp.int32), QUARTERS)

    grid_spec = pltpu.PrefetchScalarGridSpec(
        num_scalar_prefetch=1,
        grid=(QUARTERS, N_HALVES),
        in_specs=[
            pl.BlockSpec((K, BLK), lambda k, n, order: (0, 0)),
            pl.BlockSpec(
                (K_STEP, N_TILE), lambda k, n, order: (order[k], n)
            ),
        ],
        out_specs=pl.BlockSpec((BLK, N), lambda k, n, order: (0, 0)),
        scratch_shapes=[
            pltpu.VMEM((N_DEV, BLK, BLK), jnp.float32),
            pltpu.VMEM((BLK, K_STEP), jnp.float32),
            pltpu.VMEM((N_HALVES, BLK, N_TILE), jnp.float32),
            pltpu.VMEM((1, BLK), jnp.float32),
            pltpu.VMEM((N_DEV, BLK), jnp.float32),
            pltpu.SemaphoreType.DMA((N_DEV,)),
            pltpu.SemaphoreType.DMA((N_DEV,)),
            pltpu.SemaphoreType.DMA((N_DEV,)),
            pltpu.SemaphoreType.DMA((N_DEV,)),
        ],
    )

    return pl.pallas_call(
        body,
        grid_spec=grid_spec,
        out_shape=jax.ShapeDtypeStruct((BLK, N), jnp.float32),
        compiler_params=pltpu.CompilerParams(
            collective_id=0,
            vmem_limit_bytes=56 * 1024 * 1024,
        ),
    )(order, x, w_mat)
